# baseline (speedup 1.0000x reference)
import dataclasses
import functools

import jax
import jax.numpy as jnp
from jax import lax
from jax.experimental import pallas as pl
from jax.experimental.pallas import tpu as pltpu
from jax.experimental.pallas import tpu_sc as plsc

N_NODES = 10000
N_EDGES = 160000
IN_DIM = 256
H2 = 128
HALF = 128
NC = 2
NS = 16
L = 16

E_PER_SUB = N_EDGES // NS
ECH = 128
N_CHUNK = E_PER_SUB // ECH
TAIL = E_PER_SUB - N_CHUNK * ECH
ROW_SLICE = 624

HCH = 2000


def _sc_mesh():
    return plsc.VectorSubcoreMesh(core_axis_name="c", subcore_axis_name="s")


def _sc_compiler_params():
    cp = pltpu.CompilerParams()
    if "needs_layout_passes" in pltpu.CompilerParams.__dataclass_fields__:
        cp = dataclasses.replace(cp, needs_layout_passes=False)
    return cp


HCH = 1280
H_E_SUB = 10240
PK_ROWS_CH = HCH // ECH


def _sc_hist(src, dst):
    @functools.partial(
        pl.kernel,
        mesh=_sc_mesh(),
        out_type=[
            jax.ShapeDtypeStruct((N_NODES,), jnp.float32),
            jax.ShapeDtypeStruct((N_NODES,), jnp.float32),
            jax.ShapeDtypeStruct((E_ROWS_PAD, ECH), jnp.int32),
        ],
        scratch_types=[
            pltpu.VMEM((N_NODES,), jnp.float32),
            pltpu.VMEM((HCH,), jnp.int32),
            pltpu.VMEM((HCH,), jnp.int32),
            pltpu.VMEM((R_SLICE, ECH), jnp.int32),
            pltpu.VMEM((640,), jnp.float32),
            pltpu.VMEM((640,), jnp.float32),
            pltpu.VMEM_SHARED((NS * N_NODES,), jnp.float32),
        ],
        compiler_params=_sc_compiler_params(),
    )
    def k(src_hbm, dst_hbm, dego_hbm, degi_hbm, pk_hbm,
          hist_v, sbuf, dbuf, pkbuf, tbuf, abuf, shared_h):
        c = lax.axis_index("c")
        s = lax.axis_index("s")
        zeros = jnp.zeros((L,), jnp.float32)
        ones = jnp.ones((L,), jnp.float32)
        nck = jnp.where(s < NS - 1, H_E_SUB // HCH, 5)

        @pl.loop(0, N_NODES // L)
        def _(i):
            hist_v[pl.ds(i * L, L)] = zeros

        @pl.loop(0, H_E_SUB // HCH)
        def _(kk):
            @pl.when(kk < nck)
            def _():
                base = s * H_E_SUB + kk * HCH

                @pl.when(c == 0)
                def _():
                    pltpu.sync_copy(src_hbm.at[pl.ds(base, HCH)], sbuf)
                    pltpu.sync_copy(dst_hbm.at[pl.ds(base, HCH)], dbuf)

                    @pl.loop(0, HCH // L)
                    def _(j):
                        iv = sbuf[pl.ds(j * L, L)]
                        plsc.addupdate_scatter(hist_v, [iv], ones)

                    @pl.loop(0, PK_ROWS_CH)
                    def _(r):
                        @pl.loop(0, ECH // L)
                        def _(j):
                            sv = sbuf[pl.ds(r * ECH + j * L, L)]
                            dv = dbuf[pl.ds(r * ECH + j * L, L)]
                            pkbuf[kk * PK_ROWS_CH + r, pl.ds(j * L, L)] = \
                                sv | (dv << 16)

                @pl.when(c == 1)
                def _():
                    pltpu.sync_copy(dst_hbm.at[pl.ds(base, HCH)], dbuf)

                    @pl.loop(0, HCH // L)
                    def _(j):
                        iv = dbuf[pl.ds(j * L, L)]
                        plsc.addupdate_scatter(hist_v, [iv], ones)

        @pl.when(c == 0)
        def _():
            pltpu.sync_copy(pkbuf, pk_hbm.at[pl.ds(s * R_SLICE, R_SLICE)])

        pltpu.sync_copy(hist_v, shared_h.at[pl.ds(s * N_NODES, N_NODES)])
        plsc.subcore_barrier()

        row0 = s * ROW_SLICE

        @pl.loop(0, 640 // L)
        def _(i):
            abuf[pl.ds(i * L, L)] = zeros

        @pl.loop(0, NS)
        def _(t):
            pltpu.sync_copy(shared_h.at[pl.ds(t * N_NODES + row0, 640)], tbuf)

            @pl.loop(0, 640 // L)
            def _(i):
                abuf[pl.ds(i * L, L)] = \
                    abuf[pl.ds(i * L, L)] + tbuf[pl.ds(i * L, L)]

        def writeout(out_ref):
            @pl.when(s < NS - 1)
            def _():
                pltpu.sync_copy(abuf.at[pl.ds(0, ROW_SLICE)],
                                out_ref.at[pl.ds(row0, ROW_SLICE)])

            @pl.when(s == NS - 1)
            def _():
                pltpu.sync_copy(abuf, out_ref.at[pl.ds(row0, 640)])

        @pl.when(c == 0)
        def _():
            writeout(dego_hbm)

        @pl.when(c == 1)
        def _():
            writeout(degi_hbm)

    return k(src, dst)


E_ROWS = N_EDGES // ECH
R_SLICE = 80
R_LAST = E_ROWS - R_SLICE * (NS - 1)
E_ROWS_PAD = R_SLICE * NS


def _sc_edge_pass(y_flat, packed):
    @functools.partial(
        pl.kernel,
        mesh=_sc_mesh(),
        out_type=jax.ShapeDtypeStruct((NC * N_NODES, HALF), jnp.float32),
        scratch_types=[
            pltpu.VMEM_SHARED((N_NODES, HALF), jnp.float32),
            pltpu.VMEM((ECH, HALF), jnp.float32),
            pltpu.VMEM((ECH, HALF), jnp.float32),
            pltpu.VMEM((R_SLICE, ECH), jnp.int32),
            pltpu.VMEM((ECH,), jnp.int32),
            pltpu.VMEM((ECH,), jnp.int32),
            pltpu.VMEM((ECH,), jnp.int32),
            pltpu.VMEM((ECH,), jnp.int32),
            pltpu.VMEM((ECH,), jnp.int32),
            pltpu.VMEM((ECH,), jnp.int32),
            pltpu.VMEM((ECH,), jnp.int32),
            pltpu.VMEM((ECH,), jnp.int32),
            pltpu.SemaphoreType.DMA,
            pltpu.SemaphoreType.DMA,
            pltpu.SemaphoreType.DMA,
        ],
    )
    def k(y_hbm, pk_hbm, out_hbm, acc, rows0, rows1, pidx_v,
          sidx_a0, didx_a0, sidx_b0, didx_b0,
          sidx_a1, didx_a1, sidx_b1, didx_b1, sem0, sem1, psem):
        c = lax.axis_index("c")
        s = lax.axis_index("s")
        off = c * N_NODES
        zeros = jnp.zeros((L,), jnp.float32)
        nrows = jnp.where(s < NS - 1, R_SLICE, R_LAST)

        pp = pltpu.async_copy(pk_hbm.at[pl.ds(s * R_SLICE, R_SLICE)],
                              pidx_v, psem)

        @pl.loop(0, ECH)
        def _(r):
            @pl.loop(0, HALF // L)
            def _(j):
                rows0[r, pl.ds(j * L, L)] = zeros

        row0 = s * ROW_SLICE

        @pl.loop(0, 4)
        def _(m):
            pltpu.sync_copy(rows0, acc.at[pl.ds(row0 + m * ECH, ECH)])

        @pl.when(s < NS - 1)
        def _():
            pltpu.sync_copy(rows0.at[pl.ds(0, ROW_SLICE - 4 * ECH)],
                            acc.at[pl.ds(row0 + 4 * ECH, ROW_SLICE - 4 * ECH)])

        @pl.when(s == NS - 1)
        def _():
            pltpu.sync_copy(rows0, acc.at[pl.ds(row0 + 4 * ECH, ECH)])

        pp.wait()
        plsc.subcore_barrier()

        def unpack(r, sidx, didx):
            @pl.loop(0, ECH // L)
            def _(j):
                p32 = pidx_v[r, pl.ds(j * L, L)]
                sidx[pl.ds(j * L, L)] = (p32 & 0xFFFF) + off
                didx[pl.ds(j * L, L)] = p32 >> 16

        def start_gather(sidx, rows, sem):
            return pltpu.async_copy(y_hbm.at[sidx], rows, sem)

        def finish(sidx, didx, rows, sem):
            pltpu.make_async_copy(y_hbm.at[sidx], rows, sem).wait()
            pltpu.sync_copy(rows, acc.at[didx], add=True)

        unpack(0, sidx_a0, didx_a0)
        unpack(1, sidx_b0, didx_b0)
        start_gather(sidx_a0, rows0, sem0)
        start_gather(sidx_b0, rows1, sem1)

        def step(ci, sets_now, sets_next, rows, sem):
            sidx_n, didx_n = sets_now
            sidx_x, didx_x = sets_next

            @pl.when(ci < nrows)
            def _():
                @pl.when(ci + 2 < nrows)
                def _():
                    unpack(ci + 2, sidx_x, didx_x)

                finish(sidx_n, didx_n, rows, sem)

                @pl.when(ci + 2 < nrows)
                def _():
                    start_gather(sidx_x, rows, sem)

        @pl.loop(0, R_SLICE // 4)
        def _(q):
            c0 = 4 * q
            step(c0, (sidx_a0, didx_a0), (sidx_a1, didx_a1), rows0, sem0)
            step(c0 + 1, (sidx_b0, didx_b0), (sidx_b1, didx_b1), rows1, sem1)
            step(c0 + 2, (sidx_a1, didx_a1), (sidx_a0, didx_a0), rows0, sem0)
            step(c0 + 3, (sidx_b1, didx_b1), (sidx_b0, didx_b0), rows1, sem1)

        plsc.subcore_barrier()

        @pl.when(s < NS - 1)
        def _():
            pltpu.sync_copy(acc.at[pl.ds(row0, ROW_SLICE)],
                            out_hbm.at[pl.ds(off + row0, ROW_SLICE)])

        @pl.when(s == NS - 1)
        def _():
            pltpu.sync_copy(acc.at[pl.ds(row0, ROW_SLICE + 16)],
                            out_hbm.at[pl.ds(off + row0, ROW_SLICE + 16)])

    return k(y_flat, packed)


_R = 1000
_NB = N_NODES // _R


def _tc_prep(dego, degi, features):
    def body(do_ref, di_ref, x_ref, y_ref, io_ref, ii_ref):
        inv_o = lax.rsqrt(jnp.maximum(do_ref[...], 1.0))
        inv_i = lax.rsqrt(jnp.maximum(di_ref[...], 1.0))
        io_ref[...] = inv_o
        ii_ref[...] = inv_i
        y = x_ref[...] * inv_o
        y_ref[0] = y[:, :HALF]
        y_ref[1] = y[:, HALF:]

    return pl.pallas_call(
        body,
        grid=(_NB,),
        in_specs=[
            pl.BlockSpec((_R, 1), lambda i: (i, 0)),
            pl.BlockSpec((_R, 1), lambda i: (i, 0)),
            pl.BlockSpec((_R, IN_DIM), lambda i: (i, 0)),
        ],
        out_specs=[
            pl.BlockSpec((2, _R, HALF), lambda i: (0, i, 0)),
            pl.BlockSpec((_R, 1), lambda i: (i, 0)),
            pl.BlockSpec((_R, 1), lambda i: (i, 0)),
        ],
        out_shape=[
            jax.ShapeDtypeStruct((2, N_NODES, HALF), jnp.float32),
            jax.ShapeDtypeStruct((N_NODES, 1), jnp.float32),
            jax.ShapeDtypeStruct((N_NODES, 1), jnp.float32),
        ],
    )(dego, degi, features)


def _tc_layer1(agg, inv_i, inv_o, W1, b1):
    def body(a_ref, ii_ref, io_ref, w_ref, b_ref, y_ref):
        a = jnp.concatenate([a_ref[0], a_ref[1]], axis=1) * ii_ref[...]
        h = jnp.maximum(
            jnp.dot(a, w_ref[...], preferred_element_type=jnp.float32)
            + b_ref[...], 0.0)
        y = h * io_ref[...]
        y_ref[0] = y[:, :HALF]
        y_ref[1] = y[:, HALF:]

    return pl.pallas_call(
        body,
        grid=(_NB,),
        in_specs=[
            pl.BlockSpec((2, _R, HALF), lambda i: (0, i, 0)),
            pl.BlockSpec((_R, 1), lambda i: (i, 0)),
            pl.BlockSpec((_R, 1), lambda i: (i, 0)),
            pl.BlockSpec((IN_DIM, IN_DIM), lambda i: (0, 0)),
            pl.BlockSpec((1, IN_DIM), lambda i: (0, 0)),
        ],
        out_specs=pl.BlockSpec((2, _R, HALF), lambda i: (0, i, 0)),
        out_shape=jax.ShapeDtypeStruct((2, N_NODES, HALF), jnp.float32),
    )(agg, inv_i, inv_o, W1, b1.reshape(1, IN_DIM))


def _tc_final(agg, inv_i, W2, b2, W3, b3, noise):
    def body(a_ref, ii_ref, w2_ref, b2_ref, w3_ref, b3_ref, nz_ref, o_ref):
        a = jnp.concatenate([a_ref[0], a_ref[1]], axis=1) * ii_ref[...]
        mean = jnp.dot(a, w2_ref[...], preferred_element_type=jnp.float32) \
            + b2_ref[...]
        log_std = jnp.dot(a, w3_ref[...], preferred_element_type=jnp.float32) \
            + b3_ref[...]
        o_ref[...] = mean + nz_ref[...] * jnp.exp(log_std)

    return pl.pallas_call(
        body,
        grid=(_NB,),
        in_specs=[
            pl.BlockSpec((2, _R, HALF), lambda i: (0, i, 0)),
            pl.BlockSpec((_R, 1), lambda i: (i, 0)),
            pl.BlockSpec((IN_DIM, H2), lambda i: (0, 0)),
            pl.BlockSpec((1, H2), lambda i: (0, 0)),
            pl.BlockSpec((IN_DIM, H2), lambda i: (0, 0)),
            pl.BlockSpec((1, H2), lambda i: (0, 0)),
            pl.BlockSpec((_R, H2), lambda i: (i, 0)),
        ],
        out_specs=pl.BlockSpec((_R, H2), lambda i: (i, 0)),
        out_shape=jax.ShapeDtypeStruct((N_NODES, H2), jnp.float32),
    )(agg, inv_i, W2, b2.reshape(1, H2), W3, b3.reshape(1, H2), noise)


def kernel(features, edge_index, W1, b1, W2, b2, W3, b3, noise):
    src = edge_index[0]
    dst = edge_index[1]

    dego, degi, packed = _sc_hist(src, dst)
    y1, inv_o, inv_i = _tc_prep(dego.reshape(N_NODES, 1),
                                degi.reshape(N_NODES, 1), features)
    agg1 = _sc_edge_pass(y1.reshape(NC * N_NODES, HALF), packed)
    y2 = _tc_layer1(agg1.reshape(NC, N_NODES, HALF), inv_i, inv_o, W1, b1)
    agg2 = _sc_edge_pass(y2.reshape(NC * N_NODES, HALF), packed)
    return _tc_final(agg2.reshape(NC, N_NODES, HALF), inv_i, W2, b2, W3, b3,
                     noise)


# speedup vs baseline: 1.1695x; 1.1695x over previous
import dataclasses
import functools

import jax
import jax.numpy as jnp
from jax import lax
from jax.experimental import pallas as pl
from jax.experimental.pallas import tpu as pltpu
from jax.experimental.pallas import tpu_sc as plsc

N_NODES = 10000
N_EDGES = 160000
IN_DIM = 256
H2 = 128
HALF = 128
NC = 2
NS = 16
L = 16

E_PER_SUB = N_EDGES // NS
ECH = 125
ROW_SLICE = 624

HCH = 2000


def _sc_mesh():
    return plsc.VectorSubcoreMesh(core_axis_name="c", subcore_axis_name="s")


def _sc_compiler_params():
    cp = pltpu.CompilerParams()
    if "needs_layout_passes" in pltpu.CompilerParams.__dataclass_fields__:
        cp = dataclasses.replace(cp, needs_layout_passes=False)
    return cp


def _sc_hist(src, dst):
    @functools.partial(
        pl.kernel,
        mesh=_sc_mesh(),
        out_type=jax.ShapeDtypeStruct((NC * NS, N_NODES), jnp.float32),
        scratch_types=[
            pltpu.VMEM((N_NODES,), jnp.float32),
            pltpu.VMEM((HCH,), jnp.int32),
        ],
        compiler_params=_sc_compiler_params(),
    )
    def k(src_hbm, dst_hbm, out_hbm, hist_v, idx_v):
        c = lax.axis_index("c")
        s = lax.axis_index("s")
        zeros = jnp.zeros((L,), jnp.float32)
        ones = jnp.ones((L,), jnp.float32)

        @pl.loop(0, N_NODES // L)
        def _(i):
            hist_v[pl.ds(i * L, L)] = zeros

        @pl.loop(0, E_PER_SUB // HCH)
        def _(kk):
            base = s * E_PER_SUB + kk * HCH

            @pl.when(c == 0)
            def _():
                pltpu.sync_copy(src_hbm.at[pl.ds(base, HCH)], idx_v)

            @pl.when(c == 1)
            def _():
                pltpu.sync_copy(dst_hbm.at[pl.ds(base, HCH)], idx_v)

            @pl.loop(0, HCH // L)
            def _(j):
                iv = idx_v[pl.ds(j * L, L)]
                plsc.addupdate_scatter(hist_v, [iv], ones)

        pltpu.sync_copy(hist_v, out_hbm.at[c * NS + s])

    return k(src, dst)


E_ROWS = N_EDGES // ECH
R_SLICE = E_ROWS // NS
E_ROWS_PAD = E_ROWS
PBLK = 8
N_BLK = R_SLICE // PBLK
DEPTH = 3


def _sc_edge_pass(y_flat, packed):
    @functools.partial(
        pl.kernel,
        mesh=_sc_mesh(),
        out_type=jax.ShapeDtypeStruct((NC * N_NODES, HALF), jnp.float32),
        scratch_types=[
            pltpu.VMEM_SHARED((N_NODES, HALF), jnp.float32),
            pltpu.VMEM((ECH, HALF), jnp.float32),
            pltpu.VMEM((ECH, HALF), jnp.float32),
            pltpu.VMEM((ECH, HALF), jnp.float32),
            pltpu.VMEM((PBLK, ECH), jnp.int32),
            pltpu.VMEM((ECH,), jnp.int32),
            pltpu.VMEM((ECH,), jnp.int32),
            pltpu.VMEM((ECH,), jnp.int32),
            pltpu.VMEM((ECH,), jnp.int32),
            pltpu.VMEM((ECH,), jnp.int32),
            pltpu.VMEM((ECH,), jnp.int32),
            pltpu.SemaphoreType.DMA,
            pltpu.SemaphoreType.DMA,
            pltpu.SemaphoreType.DMA,
            pltpu.SemaphoreType.DMA,
        ],
    )
    def k(y_hbm, pk_hbm, out_hbm, acc, rows0, rows1, rows2, pidx0,
          sidx0, didx0, sidx1, didx1, sidx2, didx2,
          gsem0, gsem1, gsem2, psem0):
        c = lax.axis_index("c")
        s = lax.axis_index("s")
        off = c * N_NODES
        zeros = jnp.zeros((L,), jnp.float32)
        pkbase = s * R_SLICE

        pltpu.sync_copy(pk_hbm.at[pl.ds(pkbase, PBLK)], pidx0)

        @pl.loop(0, ECH)
        def _(r):
            @pl.loop(0, HALF // L)
            def _(j):
                rows0[r, pl.ds(j * L, L)] = zeros

        row0 = s * ROW_SLICE

        @pl.loop(0, 4)
        def _(m):
            pltpu.sync_copy(rows0, acc.at[pl.ds(row0 + m * ECH, ECH)])

        @pl.when(s < NS - 1)
        def _():
            pltpu.sync_copy(rows0.at[pl.ds(0, ROW_SLICE - 4 * ECH)],
                            acc.at[pl.ds(row0 + 4 * ECH, ROW_SLICE - 4 * ECH)])

        @pl.when(s == NS - 1)
        def _():
            pltpu.sync_copy(rows0, acc.at[pl.ds(row0 + 4 * ECH, ECH)])
            pltpu.sync_copy(rows0.at[pl.ds(0, 15)],
                            acc.at[pl.ds(row0 + 5 * ECH, 15)])

        plsc.subcore_barrier()

        unpack_offs = [0, 16, 32, 48, 64, 80, 96, 109]

        def unpack_from(pidx, r, sidx, didx):
            for o in unpack_offs:
                p32 = pidx[r, pl.ds(o, L)]
                sidx[pl.ds(o, L)] = (p32 & 0xFFFF) + off
                didx[pl.ds(o, L)] = p32 >> 16

        def start_gather(sidx, rows, sem):
            pltpu.async_copy(y_hbm.at[sidx], rows, sem)

        def prep_and_start(cu, sidx, didx, rows, sem):
            blk = cu // PBLK
            r_local = cu - blk * PBLK

            @pl.when(cu % PBLK == 0)
            def _():
                pltpu.make_async_copy(
                    pk_hbm.at[pl.ds(pkbase + blk * PBLK, PBLK)],
                    pidx0, psem0).wait()

            unpack_from(pidx0, r_local, sidx, didx)

            @pl.when(cu % PBLK == PBLK - 1)
            def _():
                @pl.when(blk < N_BLK - 1)
                def _():
                    pltpu.async_copy(
                        pk_hbm.at[pl.ds(pkbase + (blk + 1) * PBLK, PBLK)],
                        pidx0, psem0)

            start_gather(sidx, rows, sem)

        def step(ci, sidx, didx, rows, sem):
            @pl.when(ci < R_SLICE)
            def _():
                pltpu.make_async_copy(y_hbm.at[sidx], rows, sem).wait()
                pltpu.sync_copy(rows, acc.at[didx], add=True)

                @pl.when(ci + DEPTH < R_SLICE)
                def _():
                    prep_and_start(ci + DEPTH, sidx, didx, rows, sem)

        for ci, (sx, dx, rw, sm) in enumerate(
                [(sidx0, didx0, rows0, gsem0),
                 (sidx1, didx1, rows1, gsem1),
                 (sidx2, didx2, rows2, gsem2)]):
            unpack_from(pidx0, ci, sx, dx)
            start_gather(sx, rw, sm)

        @pl.loop(0, (R_SLICE + DEPTH - 1) // DEPTH)
        def _(q):
            c0 = DEPTH * q
            step(c0, sidx0, didx0, rows0, gsem0)
            step(c0 + 1, sidx1, didx1, rows1, gsem1)
            step(c0 + 2, sidx2, didx2, rows2, gsem2)

        plsc.subcore_barrier()

        @pl.when(s < NS - 1)
        def _():
            pltpu.sync_copy(acc.at[pl.ds(row0, ROW_SLICE)],
                            out_hbm.at[pl.ds(off + row0, ROW_SLICE)])

        @pl.when(s == NS - 1)
        def _():
            pltpu.sync_copy(acc.at[pl.ds(row0, ROW_SLICE + 16)],
                            out_hbm.at[pl.ds(off + row0, ROW_SLICE + 16)])

    return k(y_flat, packed)


_R = 1000
_NB = N_NODES // _R


def _tc_prep(hists, features):
    def body(h_ref, x_ref, y_ref, io_ref, ii_ref):
        h = h_ref[...]
        inv_o = lax.rsqrt(jnp.maximum(jnp.sum(h[:, 0:NS], axis=1), 1.0))
        inv_i = lax.rsqrt(jnp.maximum(jnp.sum(h[:, NS:], axis=1), 1.0))
        io_ref[...] = inv_o[:, None]
        ii_ref[...] = inv_i[:, None]
        y = x_ref[...] * inv_o[:, None]
        y_ref[0] = y[:, :HALF]
        y_ref[1] = y[:, HALF:]

    return pl.pallas_call(
        body,
        grid=(_NB,),
        in_specs=[
            pl.BlockSpec((_R, NC * NS), lambda i: (i, 0)),
            pl.BlockSpec((_R, IN_DIM), lambda i: (i, 0)),
        ],
        out_specs=[
            pl.BlockSpec((2, _R, HALF), lambda i: (0, i, 0)),
            pl.BlockSpec((_R, 1), lambda i: (i, 0)),
            pl.BlockSpec((_R, 1), lambda i: (i, 0)),
        ],
        out_shape=[
            jax.ShapeDtypeStruct((2, N_NODES, HALF), jnp.float32),
            jax.ShapeDtypeStruct((N_NODES, 1), jnp.float32),
            jax.ShapeDtypeStruct((N_NODES, 1), jnp.float32),
        ],
    )(hists, features)


def _tc_layer1(agg, inv_i, inv_o, W1, b1):
    def body(a_ref, ii_ref, io_ref, w_ref, b_ref, y_ref):
        a = jnp.concatenate([a_ref[0], a_ref[1]], axis=1) * ii_ref[...]
        h = jnp.maximum(
            jnp.dot(a, w_ref[...], preferred_element_type=jnp.float32)
            + b_ref[...], 0.0)
        y = h * io_ref[...]
        y_ref[0] = y[:, :HALF]
        y_ref[1] = y[:, HALF:]

    return pl.pallas_call(
        body,
        grid=(_NB,),
        in_specs=[
            pl.BlockSpec((2, _R, HALF), lambda i: (0, i, 0)),
            pl.BlockSpec((_R, 1), lambda i: (i, 0)),
            pl.BlockSpec((_R, 1), lambda i: (i, 0)),
            pl.BlockSpec((IN_DIM, IN_DIM), lambda i: (0, 0)),
            pl.BlockSpec((1, IN_DIM), lambda i: (0, 0)),
        ],
        out_specs=pl.BlockSpec((2, _R, HALF), lambda i: (0, i, 0)),
        out_shape=jax.ShapeDtypeStruct((2, N_NODES, HALF), jnp.float32),
    )(agg, inv_i, inv_o, W1, b1.reshape(1, IN_DIM))


def _tc_final(agg, inv_i, W2, b2, W3, b3, noise):
    def body(a_ref, ii_ref, w2_ref, b2_ref, w3_ref, b3_ref, nz_ref, o_ref):
        a = jnp.concatenate([a_ref[0], a_ref[1]], axis=1) * ii_ref[...]
        mean = jnp.dot(a, w2_ref[...], preferred_element_type=jnp.float32) \
            + b2_ref[...]
        log_std = jnp.dot(a, w3_ref[...], preferred_element_type=jnp.float32) \
            + b3_ref[...]
        o_ref[...] = mean + nz_ref[...] * jnp.exp(log_std)

    return pl.pallas_call(
        body,
        grid=(_NB,),
        in_specs=[
            pl.BlockSpec((2, _R, HALF), lambda i: (0, i, 0)),
            pl.BlockSpec((_R, 1), lambda i: (i, 0)),
            pl.BlockSpec((IN_DIM, H2), lambda i: (0, 0)),
            pl.BlockSpec((1, H2), lambda i: (0, 0)),
            pl.BlockSpec((IN_DIM, H2), lambda i: (0, 0)),
            pl.BlockSpec((1, H2), lambda i: (0, 0)),
            pl.BlockSpec((_R, H2), lambda i: (i, 0)),
        ],
        out_specs=pl.BlockSpec((_R, H2), lambda i: (i, 0)),
        out_shape=jax.ShapeDtypeStruct((N_NODES, H2), jnp.float32),
    )(agg, inv_i, W2, b2.reshape(1, H2), W3, b3.reshape(1, H2), noise)


def kernel(features, edge_index, W1, b1, W2, b2, W3, b3, noise):
    src = edge_index[0]
    dst = edge_index[1]
    packed = jnp.pad((src | (dst << 16)).reshape(E_ROWS, ECH),
                     ((0, E_ROWS_PAD - E_ROWS), (0, 0)))

    hists = _sc_hist(src, dst)
    y1, inv_o, inv_i = _tc_prep(hists.T, features)
    agg1 = _sc_edge_pass(y1.reshape(NC * N_NODES, HALF), packed)
    y2 = _tc_layer1(agg1.reshape(NC, N_NODES, HALF), inv_i, inv_o, W1, b1)
    agg2 = _sc_edge_pass(y2.reshape(NC * N_NODES, HALF), packed)
    return _tc_final(agg2.reshape(NC, N_NODES, HALF), inv_i, W2, b2, W3, b3,
                     noise)
